# Initial kernel scaffold; baseline (speedup 1.0000x reference)
import jax
import jax.numpy as jnp
from jax.experimental import pallas as pl


def kernel(fst_rep, state_table, token_table, final_table, W, b):
    raise NotImplementedError("write your pallas kernel here")


# speedup vs baseline: 13.1555x; 13.1555x over previous
import functools

import jax
import jax.numpy as jnp
from jax import lax
from jax.experimental import pallas as pl
from jax.experimental.pallas import tpu as pltpu
from jax.experimental.pallas import tpu_sc as plsc

B, L = 1024, 200
N = B * L
STATE_DIM = 32
TOK_DIM = 32
FIN_DIM = 16
TRAFO_DIM = 128

NC, NS = 2, 16
NW = NC * NS
IDX_LANES = 128
CH = 256
IDX_ROWS = CH // IDX_LANES
STEPS = N // (CH * NW)

_mesh = plsc.VectorSubcoreMesh(core_axis_name="c", subcore_axis_name="s")


@functools.partial(
    pl.kernel,
    mesh=_mesh,
    compiler_params=pltpu.CompilerParams(use_tc_tiling_on_sc=False),
    out_type=(
        jax.ShapeDtypeStruct((N, STATE_DIM), jnp.float32),
        jax.ShapeDtypeStruct((N, STATE_DIM), jnp.float32),
        jax.ShapeDtypeStruct((N, TOK_DIM), jnp.float32),
        jax.ShapeDtypeStruct((N, TOK_DIM), jnp.float32),
        jax.ShapeDtypeStruct((N, FIN_DIM), jnp.float32),
    ),
    scratch_types=[
        pltpu.VMEM((5, IDX_ROWS, IDX_LANES), jnp.int32),
        pltpu.VMEM((CH, STATE_DIM), jnp.float32),
        pltpu.VMEM((CH, STATE_DIM), jnp.float32),
        pltpu.VMEM((CH, TOK_DIM), jnp.float32),
        pltpu.VMEM((CH, TOK_DIM), jnp.float32),
        pltpu.VMEM((CH, FIN_DIM), jnp.float32),
        pltpu.SemaphoreType.DMA,
    ],
)
def _sc_gather(idx_hbm, state_hbm, token_hbm, final_hbm,
               g_from, g_to, g_in, g_out, g_fin,
               idx_v, r_from, r_to, r_in, r_out, r_fin, sem):
    wid = lax.axis_index("s") * NC + lax.axis_index("c")

    def step(k, carry):
        chunk = wid + k * NW
        base = chunk * CH
        row = chunk * IDX_ROWS
        for col in range(5):
            pltpu.sync_copy(idx_hbm.at[col, pl.ds(row, IDX_ROWS)],
                            idx_v.at[col])
        copies = []
        for col, tbl, rbuf in (
                (0, state_hbm, r_from),
                (3, state_hbm, r_to),
                (1, token_hbm, r_in),
                (2, token_hbm, r_out),
                (4, final_hbm, r_fin)):
            for j in range(IDX_ROWS):
                copies.append(pltpu.async_copy(
                    tbl.at[idx_v.at[col, j]],
                    rbuf.at[pl.ds(j * IDX_LANES, IDX_LANES)], sem))
        for c in copies:
            c.wait()
        pltpu.sync_copy(r_from, g_from.at[pl.ds(base, CH)])
        pltpu.sync_copy(r_to, g_to.at[pl.ds(base, CH)])
        pltpu.sync_copy(r_in, g_in.at[pl.ds(base, CH)])
        pltpu.sync_copy(r_out, g_out.at[pl.ds(base, CH)])
        pltpu.sync_copy(r_fin, g_fin.at[pl.ds(base, CH)])
        return carry

    lax.fori_loop(0, STEPS, step, 0)


M_BLK = 2048


def _mm_body(g0, g1, g2, g3, g4, w_ref, b_ref, o_ref):
    acc = jnp.dot(g0[...], w_ref[0:32, :], preferred_element_type=jnp.float32)
    acc += jnp.dot(g1[...], w_ref[32:64, :], preferred_element_type=jnp.float32)
    acc += jnp.dot(g2[...], w_ref[64:96, :], preferred_element_type=jnp.float32)
    acc += jnp.dot(g3[...], w_ref[96:128, :], preferred_element_type=jnp.float32)
    acc += jnp.dot(g4[...], w_ref[128:144, :], preferred_element_type=jnp.float32)
    o_ref[...] = acc + b_ref[...]


_matmul = pl.pallas_call(
    _mm_body,
    grid=(N // M_BLK,),
    in_specs=[
        pl.BlockSpec((M_BLK, STATE_DIM), lambda i: (i, 0)),
        pl.BlockSpec((M_BLK, STATE_DIM), lambda i: (i, 0)),
        pl.BlockSpec((M_BLK, TOK_DIM), lambda i: (i, 0)),
        pl.BlockSpec((M_BLK, TOK_DIM), lambda i: (i, 0)),
        pl.BlockSpec((M_BLK, FIN_DIM), lambda i: (i, 0)),
        pl.BlockSpec((144, TRAFO_DIM), lambda i: (0, 0)),
        pl.BlockSpec((1, TRAFO_DIM), lambda i: (0, 0)),
    ],
    out_specs=pl.BlockSpec((M_BLK, TRAFO_DIM), lambda i: (i, 0)),
    out_shape=jax.ShapeDtypeStruct((N, TRAFO_DIM), jnp.float32),
)


def kernel(fst_rep, state_table, token_table, final_table, W, b):
    idx = fst_rep.reshape(N, 5).T.reshape(5, N // IDX_LANES, IDX_LANES)
    g_from, g_to, g_in, g_out, g_fin = _sc_gather(
        idx, state_table, token_table, final_table)
    out = _matmul(g_from, g_to, g_in, g_out, g_fin, W,
                  b.reshape(1, TRAFO_DIM))
    return out.reshape(B, L, TRAFO_DIM)


# speedup vs baseline: 16.6777x; 1.2677x over previous
import functools

import jax
import jax.numpy as jnp
from jax import lax
from jax.experimental import pallas as pl
from jax.experimental.pallas import tpu as pltpu
from jax.experimental.pallas import tpu_sc as plsc

B, L = 1024, 200
N = B * L
STATE_DIM = 32
TOK_DIM = 32
FIN_DIM = 16
TRAFO_DIM = 128

NC, NS = 2, 16
NW = NC * NS
NTILE = N // NW
CH = 256
IDX_LANES = 128
IDX_ROWS = CH // IDX_LANES
STEPS = NTILE // CH
IDX_ROWS_TILE = NTILE // IDX_LANES

_mesh = plsc.VectorSubcoreMesh(core_axis_name="c", subcore_axis_name="s")

_COLS = (0, 3, 1, 2)


@functools.partial(
    pl.kernel,
    mesh=_mesh,
    compiler_params=pltpu.CompilerParams(use_tc_tiling_on_sc=False),
    out_type=(
        jax.ShapeDtypeStruct((N, 4 * STATE_DIM), jnp.float32),
        jax.ShapeDtypeStruct((N, FIN_DIM), jnp.float32),
    ),
    scratch_types=(
        [pltpu.VMEM((5, IDX_ROWS_TILE, IDX_LANES), jnp.int32)]
        + [pltpu.VMEM((CH, STATE_DIM), jnp.float32) for _ in range(4)]
        + [pltpu.VMEM((CH, FIN_DIM), jnp.float32)]
        + [pltpu.SemaphoreType.DMA]
    ),
)
def _sc_gather(idx_hbm, state_hbm, token_hbm, final_hbm,
               g4x, g_fin,
               idx_v, r0, r1, r2, r3, r_fin, sem):
    wid = lax.axis_index("s") * NC + lax.axis_index("c")
    tbase = wid * NTILE
    trow = wid * IDX_ROWS_TILE

    tables = (state_hbm, state_hbm, token_hbm, token_hbm)
    rbufs = (r0, r1, r2, r3)

    for col in range(5):
        pltpu.sync_copy(idx_hbm.at[col, pl.ds(trow, IDX_ROWS_TILE)],
                        idx_v.at[col])

    def step(k, carry):
        base = tbase + k * CH
        row = k * IDX_ROWS
        copies = []
        for t in range(4):
            for j in range(IDX_ROWS):
                copies.append(pltpu.async_copy(
                    tables[t].at[idx_v.at[_COLS[t], row + j]],
                    rbufs[t].at[pl.ds(j * IDX_LANES, IDX_LANES)], sem))
        for j in range(IDX_ROWS):
            copies.append(pltpu.async_copy(
                final_hbm.at[idx_v.at[4, row + j]],
                r_fin.at[pl.ds(j * IDX_LANES, IDX_LANES)], sem))
        for c in copies:
            c.wait()
        for t in range(4):
            pltpu.sync_copy(
                rbufs[t],
                g4x.at[pl.ds(base, CH), pl.ds(t * STATE_DIM, STATE_DIM)])
        pltpu.sync_copy(r_fin, g_fin.at[pl.ds(base, CH)])
        return carry

    lax.fori_loop(0, STEPS, step, 0)


M_BLK = 2048


def _mm_body(g4x, gfin, w_ref, b_ref, o_ref):
    acc = jnp.dot(g4x[...], w_ref[0:128, :], preferred_element_type=jnp.float32)
    acc += jnp.dot(gfin[...], w_ref[128:144, :], preferred_element_type=jnp.float32)
    o_ref[...] = acc + b_ref[...]


_matmul = pl.pallas_call(
    _mm_body,
    grid=(N // M_BLK,),
    in_specs=[
        pl.BlockSpec((M_BLK, 4 * STATE_DIM), lambda i: (i, 0)),
        pl.BlockSpec((M_BLK, FIN_DIM), lambda i: (i, 0)),
        pl.BlockSpec((144, TRAFO_DIM), lambda i: (0, 0)),
        pl.BlockSpec((1, TRAFO_DIM), lambda i: (0, 0)),
    ],
    out_specs=pl.BlockSpec((M_BLK, TRAFO_DIM), lambda i: (i, 0)),
    out_shape=jax.ShapeDtypeStruct((N, TRAFO_DIM), jnp.float32),
)


def kernel(fst_rep, state_table, token_table, final_table, W, b):
    idx = fst_rep.reshape(N, 5).T.reshape(5, N // IDX_LANES, IDX_LANES)
    g4x, g_fin = _sc_gather(idx, state_table, token_table, final_table)
    out = _matmul(g4x, g_fin, W, b.reshape(1, TRAFO_DIM))
    return out.reshape(B, L, TRAFO_DIM)
